# Initial kernel scaffold; baseline (speedup 1.0000x reference)
import jax
import jax.numpy as jnp
from jax.experimental import pallas as pl


def kernel(x, x1, edge_index, edge_index1, gcn1_W, gcn1_b, gcn2_W, gcn2_b, sage1_Wl, sage1_bl, sage1_Wr, sage2_Wl, sage2_bl, sage2_Wr, kan1_base_w, kan1_spline_w, kan1_scaler, kan2_base_w, kan2_spline_w, kan2_scaler):
    raise NotImplementedError("write your pallas kernel here")


# speedup vs baseline: 7.8794x; 7.8794x over previous
import functools

import jax
import jax.numpy as jnp
import numpy as np
from jax import lax
from jax.experimental import pallas as pl
from jax.experimental.pallas import tpu as pltpu
from jax.experimental.pallas import tpu_sc as plsc

N = 10000
E = 160000
D_IN = 256
H = 512
K = H // 2
GRID_SIZE = 5
SPLINE_ORDER = 3

NPAD = 10240
NTRASH = 240
CHUNK = 128
EPC = 1264
EP = EPC * CHUNK
CPS = EPC // 16
RPS = NPAD // 16
BLK = 1024
NBLK = NPAD // BLK

_G = (np.arange(-SPLINE_ORDER, GRID_SIZE + SPLINE_ORDER + 1, dtype=np.float32)
      * np.float32(2.0 / GRID_SIZE) - np.float32(1.0))

def _sc_mesh():
    return plsc.VectorSubcoreMesh(core_axis_name="c", subcore_axis_name="s")


@jax.jit
def _sc_counts(dsts2, ones_blk, zeros_blk):
    @functools.partial(
        pl.kernel,
        out_type=jax.ShapeDtypeStruct((2 * NPAD, 128), jnp.float32),
        mesh=_sc_mesh(),
        scratch_types=[
            pltpu.VMEM((CHUNK,), jnp.int32),
            pltpu.VMEM((CHUNK, 128), jnp.float32),
            pltpu.VMEM_SHARED((NPAD, 128), jnp.float32),
            pltpu.SemaphoreType.DMA,
        ],
    )
    def kern(dsts_hbm, ones_hbm, zeros_hbm, out_hbm, dst_v, ones_v, acc, sem):
        c = lax.axis_index("c")
        s = lax.axis_index("s")
        pltpu.sync_copy(zeros_hbm, acc.at[pl.ds(s * RPS, RPS)])
        pltpu.sync_copy(ones_hbm, ones_v)
        plsc.subcore_barrier()

        @pl.loop(0, CPS)
        def _(j):
            r = c * EPC + s * CPS + j
            pltpu.sync_copy(dsts_hbm.at[r], dst_v)
            pltpu.sync_copy(ones_v, acc.at[dst_v], add=True)

        plsc.subcore_barrier()
        pltpu.sync_copy(acc.at[pl.ds(s * RPS, RPS)],
                        out_hbm.at[pl.ds(c * NPAD + s * RPS, RPS)])

    return kern(dsts2, ones_blk, zeros_blk)


@jax.jit
def _sc_segsum(table, srcs2, dsts, zeros_blk):
    @functools.partial(
        pl.kernel,
        out_type=jax.ShapeDtypeStruct((2 * NPAD, 128), jnp.float32),
        mesh=_sc_mesh(),
        scratch_types=[
            pltpu.VMEM((CHUNK,), jnp.int32),
            pltpu.VMEM((CHUNK,), jnp.int32),
            pltpu.VMEM((CHUNK, 128), jnp.float32),
            pltpu.VMEM_SHARED((NPAD, 128), jnp.float32),
            pltpu.SemaphoreType.DMA,
        ],
    )
    def kern(table_hbm, srcs_hbm, dsts_hbm, zeros_hbm, out_hbm,
             src_v, dst_v, rows_v, acc, sem):
        c = lax.axis_index("c")
        s = lax.axis_index("s")
        pltpu.sync_copy(zeros_hbm, acc.at[pl.ds(s * RPS, RPS)])
        plsc.subcore_barrier()

        @pl.loop(0, CPS)
        def _(j):
            r = s * CPS + j
            pltpu.sync_copy(srcs_hbm.at[c * EPC + r], src_v)
            pltpu.sync_copy(dsts_hbm.at[r], dst_v)
            pltpu.async_copy(table_hbm.at[src_v], rows_v, sem).wait()
            pltpu.sync_copy(rows_v, acc.at[dst_v], add=True)

        plsc.subcore_barrier()
        pltpu.sync_copy(acc.at[pl.ds(s * RPS, RPS)],
                        out_hbm.at[pl.ds(c * NPAD + s * RPS, RPS)])

    return kern(table, srcs2, dsts, zeros_blk)


def _scale_body(x_ref, cnt_ref, out_ref):
    dinv = lax.rsqrt(cnt_ref[:, 0:1] + 1.0)
    xs = x_ref[...] * dinv
    out_ref[0] = xs[:, :128]
    out_ref[1] = xs[:, 128:]


@jax.jit
def _tc_scale(xp, cnt):
    return pl.pallas_call(
        _scale_body,
        grid=(NBLK,),
        in_specs=[
            pl.BlockSpec((BLK, 256), lambda i: (i, 0)),
            pl.BlockSpec((BLK, 16), lambda i: (i, 0)),
        ],
        out_specs=pl.BlockSpec((2, BLK, 128), lambda i: (0, i, 0)),
        out_shape=jax.ShapeDtypeStruct((2, NPAD, 128), jnp.float32),
    )(xp, cnt)


def _cat_halves(st):
    return jnp.concatenate([st[0], st[1]], axis=1)


def _dot(a, b):
    return jnp.dot(a, b, precision=jax.lax.Precision.HIGHEST)


def _dot_t(a, w):
    return lax.dot_general(a, w, (((1,), (1,)), ((), ())),
                           precision=jax.lax.Precision.HIGHEST)


def _dense_body(s0_ref, xs0_ref, s1_ref, xs1_ref, cnt0_ref, cnt1_ref,
                g1w_ref, g1b_ref, g2w_ref, g2b_ref,
                s1wl_ref, s1wr_ref, s1bl_ref, s2wl_ref, s2wr_ref, s2bl_ref,
                ya_ref, yb_ref, ra_ref, rb_ref):
    dinv0 = lax.rsqrt(cnt0_ref[:, 0:1] + 1.0)
    dinv1 = lax.rsqrt(cnt1_ref[:, 0:1] + 1.0)
    agg0 = dinv0 * (_cat_halves(s0_ref[...]) + _cat_halves(xs0_ref[...]))
    agg1 = dinv1 * (_cat_halves(s1_ref[...]) + _cat_halves(xs1_ref[...]))
    h0a = jax.nn.relu(_dot(agg0, g1w_ref[...]) + g1b_ref[...])
    h1c = _dot(agg1, g2w_ref[...]) + g2b_ref[...]
    h1 = jax.nn.relu(h1c)
    h0 = h0a + h1
    bpre = h1 + h1c

    ya = _dot_t(h0, s1wl_ref[...])
    yb = _dot_t(bpre, s2wl_ref[...])
    ya_ref[0], ya_ref[1] = ya[:, :128], ya[:, 128:]
    yb_ref[0], yb_ref[1] = yb[:, :128], yb[:, 128:]
    ra_ref[...] = _dot_t(h0, s1wr_ref[...]) + s1bl_ref[...]
    rb_ref[...] = _dot_t(bpre, s2wr_ref[...]) + s2bl_ref[...]


@jax.jit
def _tc_dense(s0, xs0, s1, xs1, cnt0, cnt1, g1w, g1b, g2w, g2b,
              s1wl, s1wr, s1bl, s2wl, s2wr, s2bl):
    st_spec = pl.BlockSpec((2, BLK, 128), lambda i: (0, i, 0))
    cnt_spec = pl.BlockSpec((BLK, 16), lambda i: (i, 0))
    full = lambda shape: pl.BlockSpec(shape, lambda i: tuple(0 for _ in shape))
    return pl.pallas_call(
        _dense_body,
        grid=(NBLK,),
        in_specs=[st_spec, st_spec, st_spec, st_spec, cnt_spec, cnt_spec,
                  full((256, 512)), full((1, 512)), full((256, 512)),
                  full((1, 512)),
                  full((256, 512)), full((256, 512)), full((1, 256)),
                  full((256, 512)), full((256, 512)), full((1, 256))],
        out_specs=[st_spec, st_spec,
                   pl.BlockSpec((BLK, 256), lambda i: (i, 0)),
                   pl.BlockSpec((BLK, 256), lambda i: (i, 0))],
        out_shape=[jax.ShapeDtypeStruct((2, NPAD, 128), jnp.float32),
                   jax.ShapeDtypeStruct((2, NPAD, 128), jnp.float32),
                   jax.ShapeDtypeStruct((NPAD, 256), jnp.float32),
                   jax.ShapeDtypeStruct((NPAD, 256), jnp.float32)],
    )(s0, xs0, s1, xs1, cnt0, cnt1, g1w, g1b, g2w, g2b,
      s1wl, s1wr, s1bl, s2wl, s2wr, s2bl)


def _silu(x):
    return x / (1.0 + jnp.exp(-x))


def _basis8(x):
    b = [jnp.where((x >= _G[j]) & (x < _G[j + 1]), 1.0, 0.0) for j in range(11)]
    for k in range(1, SPLINE_ORDER + 1):
        nb = []
        for j in range(11 - k):
            linv = np.float32(1.0) / (_G[j + k] - _G[j])
            rinv = np.float32(1.0) / (_G[j + k + 1] - _G[j + 1])
            nb.append((x - _G[j]) * linv * b[j]
                      + (_G[j + k + 1] - x) * rinv * b[j + 1])
        b = nb
    return b


def _kan_scalar(z, k1bw_ref, sc1_ref, k2bw_ref, w2_ref):
    b1 = _basis8(z)
    z1 = _dot(_silu(z), k1bw_ref[...])
    for c in range(8):
        z1 = z1 + _dot(b1[c], sc1_ref[c])
    b2 = _basis8(z1)
    out = jnp.sum(_silu(z1) * k2bw_ref[...], axis=1)
    for c in range(8):
        out = out + jnp.sum(b2[c] * w2_ref[c:c + 1, :], axis=1)
    return out


def _kan_body(sa_ref, sb_ref, ra_ref, rb_ref, cnt0_ref, cnt1_ref,
              k1bw_ref, sc1_ref, k2bw_ref, w2_ref, out_ref):
    i = pl.program_id(0)
    cnt0c = jnp.maximum(cnt0_ref[:, 0:1], 1.0)
    cnt1c = jnp.maximum(cnt1_ref[:, 0:1], 1.0)
    a = jax.nn.relu(_cat_halves(sa_ref[...]) / cnt0c + ra_ref[...])
    bb = jax.nn.relu(_cat_halves(sb_ref[...]) / cnt1c + rb_ref[...])
    va = _kan_scalar(a, k1bw_ref, sc1_ref, k2bw_ref, w2_ref)
    vb = _kan_scalar(bb, k1bw_ref, sc1_ref, k2bw_ref, w2_ref)
    rows = i * BLK + lax.broadcasted_iota(jnp.int32, (BLK,), 0)
    valid = rows < N
    part = jnp.sum(jnp.where(valid, va + vb, 0.0))

    @pl.when(i == 0)
    def _():
        out_ref[0, 0] = 0.0

    acc = out_ref[0, 0] + part

    @pl.when(i == NBLK - 1)
    def _():
        out_ref[0, 0] = acc / np.float32(2 * N)

    @pl.when(i < NBLK - 1)
    def _():
        out_ref[0, 0] = acc


@jax.jit
def _tc_kan(sa, sb, ra, rb, cnt0, cnt1, k1bw, sc1, k2bw, w2):
    st_spec = pl.BlockSpec((2, BLK, 128), lambda i: (0, i, 0))
    cnt_spec = pl.BlockSpec((BLK, 16), lambda i: (i, 0))
    full = lambda shape: pl.BlockSpec(shape, lambda i: tuple(0 for _ in shape))
    return pl.pallas_call(
        _kan_body,
        grid=(NBLK,),
        in_specs=[st_spec, st_spec,
                  pl.BlockSpec((BLK, 256), lambda i: (i, 0)),
                  pl.BlockSpec((BLK, 256), lambda i: (i, 0)),
                  cnt_spec, cnt_spec,
                  full((256, 256)), full((8, 256, 256)),
                  full((1, 256)), full((8, 256))],
        out_specs=pl.BlockSpec((1, 1), lambda i: (0, 0),
                               memory_space=pltpu.SMEM),
        out_shape=jax.ShapeDtypeStruct((1, 1), jnp.float32),
    )(sa, sb, ra, rb, cnt0, cnt1, k1bw, sc1, k2bw, w2)


def _prep_edges(edge_index):
    src = edge_index[0].astype(jnp.int32)
    dst = edge_index[1].astype(jnp.int32)
    pad_n = EP - E
    pad_src = (jnp.arange(pad_n, dtype=jnp.int32) % N)
    pad_dst = N + (jnp.arange(pad_n, dtype=jnp.int32) % NTRASH)
    srcp = jnp.concatenate([src, pad_src])
    dstp = jnp.concatenate([dst, pad_dst])
    srcs2 = jnp.concatenate([srcp, srcp + NPAD]).reshape(2 * EPC, CHUNK)
    dsts = dstp.reshape(EPC, CHUNK)
    return srcs2, dsts


def kernel(x, x1, edge_index, edge_index1, gcn1_W, gcn1_b, gcn2_W, gcn2_b,
           sage1_Wl, sage1_bl, sage1_Wr, sage2_Wl, sage2_bl, sage2_Wr,
           kan1_base_w, kan1_spline_w, kan1_scaler,
           kan2_base_w, kan2_spline_w, kan2_scaler):
    f32 = jnp.float32
    srcs0, dsts0 = _prep_edges(edge_index)
    srcs1, dsts1 = _prep_edges(edge_index1)

    ones_blk = jnp.ones((CHUNK, 128), f32)
    zeros_blk = jnp.zeros((RPS, 128), f32)

    dsts_both = jnp.concatenate([dsts0, dsts1], axis=0)
    cnt_all = _sc_counts(dsts_both, ones_blk, zeros_blk)
    cnt0 = cnt_all[:NPAD, :16]
    cnt1 = cnt_all[NPAD:, :16]

    xp = jnp.pad(x.astype(f32), ((0, NPAD - N), (0, 0)))
    x1p = jnp.pad(x1.astype(f32), ((0, NPAD - N), (0, 0)))
    xs0 = _tc_scale(xp, cnt0)
    xs1 = _tc_scale(x1p, cnt1)

    s0 = _sc_segsum(xs0.reshape(2 * NPAD, 128), srcs0, dsts0, zeros_blk)
    s1 = _sc_segsum(xs1.reshape(2 * NPAD, 128), srcs1, dsts1, zeros_blk)

    ya, yb, ra, rb = _tc_dense(
        s0.reshape(2, NPAD, 128), xs0, s1.reshape(2, NPAD, 128), xs1,
        cnt0, cnt1, gcn1_W, gcn1_b.reshape(1, H), gcn2_W, gcn2_b.reshape(1, H),
        sage1_Wl, sage1_Wr, sage1_bl.reshape(1, K),
        sage2_Wl, sage2_Wr, sage2_bl.reshape(1, K))

    sa = _sc_segsum(ya.reshape(2 * NPAD, 128), srcs0, dsts0, zeros_blk)
    sb = _sc_segsum(yb.reshape(2 * NPAD, 128), srcs1, dsts1, zeros_blk)

    scaled1 = (kan1_spline_w * kan1_scaler[:, :, None]).reshape(K, K * 8)
    sc1 = scaled1.reshape(K, K, 8).transpose(2, 1, 0)
    w2 = (kan2_spline_w * kan2_scaler[:, :, None]).reshape(K, 8).T

    return _tc_kan(sa.reshape(2, NPAD, 128), sb.reshape(2, NPAD, 128),
                   ra, rb, cnt0, cnt1, kan1_base_w.T, sc1, kan2_base_w, w2)


# speedup vs baseline: 11.2636x; 1.4295x over previous
import functools

import jax
import jax.numpy as jnp
import numpy as np
from jax import lax
from jax.experimental import pallas as pl
from jax.experimental.pallas import tpu as pltpu
from jax.experimental.pallas import tpu_sc as plsc

N = 10000
E = 160000
D_IN = 256
H = 512
K = H // 2
GRID_SIZE = 5
SPLINE_ORDER = 3

NPAD = 10240
NTRASH = 240
CHUNK = 128
EPC = 1280
EP = EPC * CHUNK
CPS = EPC // 16
NBUF = 2
RPS = NPAD // 16
BLK = 1024
NBLK = NPAD // BLK

_G = (np.arange(-SPLINE_ORDER, GRID_SIZE + SPLINE_ORDER + 1, dtype=np.float32)
      * np.float32(2.0 / GRID_SIZE) - np.float32(1.0))

def _sc_mesh():
    return plsc.VectorSubcoreMesh(core_axis_name="c", subcore_axis_name="s")


@jax.jit
def _sc_counts(dsts2, ones_blk, zeros_blk):
    @functools.partial(
        pl.kernel,
        out_type=jax.ShapeDtypeStruct((2 * NPAD, 128), jnp.float32),
        mesh=_sc_mesh(),
        scratch_types=[
            pltpu.VMEM((CPS, CHUNK), jnp.int32),
            pltpu.VMEM((CHUNK, 128), jnp.float32),
            pltpu.VMEM_SHARED((NPAD, 128), jnp.float32),
            pltpu.SemaphoreType.DMA,
        ],
    )
    def kern(dsts_hbm, ones_hbm, zeros_hbm, out_hbm, dst_all, ones_v, acc, sem):
        c = lax.axis_index("c")
        s = lax.axis_index("s")
        pltpu.sync_copy(zeros_hbm, acc.at[pl.ds(s * RPS, RPS)])
        pltpu.sync_copy(ones_hbm, ones_v)
        pltpu.sync_copy(dsts_hbm.at[pl.ds(c * EPC + s * CPS, CPS)], dst_all)
        plsc.subcore_barrier()

        @pl.loop(0, CPS)
        def _(j):
            pltpu.sync_copy(ones_v, acc.at[dst_all.at[j]], add=True)

        plsc.subcore_barrier()
        pltpu.sync_copy(acc.at[pl.ds(s * RPS, RPS)],
                        out_hbm.at[pl.ds(c * NPAD + s * RPS, RPS)])

    return kern(dsts2, ones_blk, zeros_blk)


@jax.jit
def _sc_segsum(table, srcs2, dsts, zeros_blk):
    @functools.partial(
        pl.kernel,
        out_type=jax.ShapeDtypeStruct((2 * NPAD, 128), jnp.float32),
        mesh=_sc_mesh(),
        scratch_types=[
            pltpu.VMEM((CPS, CHUNK), jnp.int32),
        ] + [pltpu.VMEM((CHUNK,), jnp.int32) for _ in range(NBUF)]
          + [pltpu.VMEM((CHUNK, 128), jnp.float32) for _ in range(NBUF)]
          + [pltpu.SemaphoreType.DMA for _ in range(NBUF)]
          + [pltpu.VMEM_SHARED((NPAD, 128), jnp.float32)],
    )
    def kern(table_hbm, srcs_hbm, dsts_hbm, zeros_hbm, out_hbm,
             dst_all, *rest):
        src_v = rest[:NBUF]
        rows = rest[NBUF:2 * NBUF]
        sems = rest[2 * NBUF:3 * NBUF]
        acc = rest[3 * NBUF]
        c = lax.axis_index("c")
        s = lax.axis_index("s")
        pltpu.sync_copy(zeros_hbm, acc.at[pl.ds(s * RPS, RPS)])
        pltpu.sync_copy(dsts_hbm.at[pl.ds(s * CPS, CPS)], dst_all)
        plsc.subcore_barrier()

        for b in range(NBUF):
            pltpu.sync_copy(srcs_hbm.at[c * EPC + s * CPS + b], src_v[b])
            pltpu.async_copy(table_hbm.at[src_v[b]], rows[b], sems[b])

        @pl.loop(0, CPS // NBUF)
        def _(i):
            for b in range(NBUF):
                j = i * NBUF + b
                pltpu.make_async_copy(
                    table_hbm.at[src_v[b]], rows[b], sems[b]).wait()
                pltpu.sync_copy(rows[b], acc.at[dst_all.at[j]], add=True)

                @pl.when(j + NBUF < CPS)
                def _():
                    pltpu.sync_copy(
                        srcs_hbm.at[c * EPC + s * CPS + j + NBUF], src_v[b])
                    pltpu.async_copy(table_hbm.at[src_v[b]], rows[b], sems[b])

        plsc.subcore_barrier()
        pltpu.sync_copy(acc.at[pl.ds(s * RPS, RPS)],
                        out_hbm.at[pl.ds(c * NPAD + s * RPS, RPS)])

    return kern(table, srcs2, dsts, zeros_blk)


def _scale_body(x_ref, cnt_ref, out_ref):
    dinv = lax.rsqrt(cnt_ref[:, 0:1] + 1.0)
    xs = x_ref[...] * dinv
    out_ref[0] = xs[:, :128]
    out_ref[1] = xs[:, 128:]


@jax.jit
def _tc_scale(xp, cnt):
    return pl.pallas_call(
        _scale_body,
        grid=(NBLK,),
        in_specs=[
            pl.BlockSpec((BLK, 256), lambda i: (i, 0)),
            pl.BlockSpec((BLK, 16), lambda i: (i, 0)),
        ],
        out_specs=pl.BlockSpec((2, BLK, 128), lambda i: (0, i, 0)),
        out_shape=jax.ShapeDtypeStruct((2, NPAD, 128), jnp.float32),
    )(xp, cnt)


def _cat_halves(st):
    return jnp.concatenate([st[0], st[1]], axis=1)


def _dot(a, b):
    return jnp.dot(a, b, precision=jax.lax.Precision.HIGHEST)


def _dot_t(a, w):
    return lax.dot_general(a, w, (((1,), (1,)), ((), ())),
                           precision=jax.lax.Precision.HIGHEST)


def _dense_body(s0_ref, xs0_ref, s1_ref, xs1_ref, cnt0_ref, cnt1_ref,
                g1w_ref, g1b_ref, g2w_ref, g2b_ref,
                s1wl_ref, s1wr_ref, s1bl_ref, s2wl_ref, s2wr_ref, s2bl_ref,
                ya_ref, yb_ref, ra_ref, rb_ref):
    dinv0 = lax.rsqrt(cnt0_ref[:, 0:1] + 1.0)
    dinv1 = lax.rsqrt(cnt1_ref[:, 0:1] + 1.0)
    agg0 = dinv0 * (_cat_halves(s0_ref[...]) + _cat_halves(xs0_ref[...]))
    agg1 = dinv1 * (_cat_halves(s1_ref[...]) + _cat_halves(xs1_ref[...]))
    h0a = jax.nn.relu(_dot(agg0, g1w_ref[...]) + g1b_ref[...])
    h1c = _dot(agg1, g2w_ref[...]) + g2b_ref[...]
    h1 = jax.nn.relu(h1c)
    h0 = h0a + h1
    bpre = h1 + h1c

    ya = _dot_t(h0, s1wl_ref[...])
    yb = _dot_t(bpre, s2wl_ref[...])
    ya_ref[0], ya_ref[1] = ya[:, :128], ya[:, 128:]
    yb_ref[0], yb_ref[1] = yb[:, :128], yb[:, 128:]
    ra_ref[...] = _dot_t(h0, s1wr_ref[...]) + s1bl_ref[...]
    rb_ref[...] = _dot_t(bpre, s2wr_ref[...]) + s2bl_ref[...]


@jax.jit
def _tc_dense(s0, xs0, s1, xs1, cnt0, cnt1, g1w, g1b, g2w, g2b,
              s1wl, s1wr, s1bl, s2wl, s2wr, s2bl):
    st_spec = pl.BlockSpec((2, BLK, 128), lambda i: (0, i, 0))
    cnt_spec = pl.BlockSpec((BLK, 16), lambda i: (i, 0))
    full = lambda shape: pl.BlockSpec(shape, lambda i: tuple(0 for _ in shape))
    return pl.pallas_call(
        _dense_body,
        grid=(NBLK,),
        in_specs=[st_spec, st_spec, st_spec, st_spec, cnt_spec, cnt_spec,
                  full((256, 512)), full((1, 512)), full((256, 512)),
                  full((1, 512)),
                  full((256, 512)), full((256, 512)), full((1, 256)),
                  full((256, 512)), full((256, 512)), full((1, 256))],
        out_specs=[st_spec, st_spec,
                   pl.BlockSpec((BLK, 256), lambda i: (i, 0)),
                   pl.BlockSpec((BLK, 256), lambda i: (i, 0))],
        out_shape=[jax.ShapeDtypeStruct((2, NPAD, 128), jnp.float32),
                   jax.ShapeDtypeStruct((2, NPAD, 128), jnp.float32),
                   jax.ShapeDtypeStruct((NPAD, 256), jnp.float32),
                   jax.ShapeDtypeStruct((NPAD, 256), jnp.float32)],
    )(s0, xs0, s1, xs1, cnt0, cnt1, g1w, g1b, g2w, g2b,
      s1wl, s1wr, s1bl, s2wl, s2wr, s2bl)


def _silu(x):
    return x / (1.0 + jnp.exp(-x))


def _basis8(x):
    b = [jnp.where((x >= _G[j]) & (x < _G[j + 1]), 1.0, 0.0) for j in range(11)]
    for k in range(1, SPLINE_ORDER + 1):
        nb = []
        for j in range(11 - k):
            linv = np.float32(1.0) / (_G[j + k] - _G[j])
            rinv = np.float32(1.0) / (_G[j + k + 1] - _G[j + 1])
            nb.append((x - _G[j]) * linv * b[j]
                      + (_G[j + k + 1] - x) * rinv * b[j + 1])
        b = nb
    return b


def _kan_scalar(z, k1bw_ref, sc1_ref, k2bw_ref, w2_ref):
    b1 = _basis8(z)
    z1 = _dot(_silu(z), k1bw_ref[...])
    for c in range(8):
        z1 = z1 + _dot(b1[c], sc1_ref[c])
    b2 = _basis8(z1)
    out = jnp.sum(_silu(z1) * k2bw_ref[...], axis=1)
    for c in range(8):
        out = out + jnp.sum(b2[c] * w2_ref[c:c + 1, :], axis=1)
    return out


def _kan_body(sa_ref, sb_ref, ra_ref, rb_ref, cnt0_ref, cnt1_ref,
              k1bw_ref, sc1_ref, k2bw_ref, w2_ref, out_ref):
    i = pl.program_id(0)
    cnt0c = jnp.maximum(cnt0_ref[:, 0:1], 1.0)
    cnt1c = jnp.maximum(cnt1_ref[:, 0:1], 1.0)
    a = jax.nn.relu(_cat_halves(sa_ref[...]) / cnt0c + ra_ref[...])
    bb = jax.nn.relu(_cat_halves(sb_ref[...]) / cnt1c + rb_ref[...])
    va = _kan_scalar(a, k1bw_ref, sc1_ref, k2bw_ref, w2_ref)
    vb = _kan_scalar(bb, k1bw_ref, sc1_ref, k2bw_ref, w2_ref)
    rows = i * BLK + lax.broadcasted_iota(jnp.int32, (BLK,), 0)
    valid = rows < N
    part = jnp.sum(jnp.where(valid, va + vb, 0.0))

    @pl.when(i == 0)
    def _():
        out_ref[0, 0] = 0.0

    acc = out_ref[0, 0] + part

    @pl.when(i == NBLK - 1)
    def _():
        out_ref[0, 0] = acc / np.float32(2 * N)

    @pl.when(i < NBLK - 1)
    def _():
        out_ref[0, 0] = acc


@jax.jit
def _tc_kan(sa, sb, ra, rb, cnt0, cnt1, k1bw, sc1, k2bw, w2):
    st_spec = pl.BlockSpec((2, BLK, 128), lambda i: (0, i, 0))
    cnt_spec = pl.BlockSpec((BLK, 16), lambda i: (i, 0))
    full = lambda shape: pl.BlockSpec(shape, lambda i: tuple(0 for _ in shape))
    return pl.pallas_call(
        _kan_body,
        grid=(NBLK,),
        in_specs=[st_spec, st_spec,
                  pl.BlockSpec((BLK, 256), lambda i: (i, 0)),
                  pl.BlockSpec((BLK, 256), lambda i: (i, 0)),
                  cnt_spec, cnt_spec,
                  full((256, 256)), full((8, 256, 256)),
                  full((1, 256)), full((8, 256))],
        out_specs=pl.BlockSpec((1, 1), lambda i: (0, 0),
                               memory_space=pltpu.SMEM),
        out_shape=jax.ShapeDtypeStruct((1, 1), jnp.float32),
    )(sa, sb, ra, rb, cnt0, cnt1, k1bw, sc1, k2bw, w2)


def _prep_edges(edge_index):
    src = edge_index[0].astype(jnp.int32)
    dst = edge_index[1].astype(jnp.int32)
    pad_n = EP - E
    pad_src = (jnp.arange(pad_n, dtype=jnp.int32) % N)
    pad_dst = N + (jnp.arange(pad_n, dtype=jnp.int32) % NTRASH)
    srcp = jnp.concatenate([src, pad_src])
    dstp = jnp.concatenate([dst, pad_dst])
    srcs2 = jnp.concatenate([srcp, srcp + NPAD]).reshape(2 * EPC, CHUNK)
    dsts = dstp.reshape(EPC, CHUNK)
    return srcs2, dsts


def kernel(x, x1, edge_index, edge_index1, gcn1_W, gcn1_b, gcn2_W, gcn2_b,
           sage1_Wl, sage1_bl, sage1_Wr, sage2_Wl, sage2_bl, sage2_Wr,
           kan1_base_w, kan1_spline_w, kan1_scaler,
           kan2_base_w, kan2_spline_w, kan2_scaler):
    f32 = jnp.float32
    srcs0, dsts0 = _prep_edges(edge_index)
    srcs1, dsts1 = _prep_edges(edge_index1)

    ones_blk = jnp.ones((CHUNK, 128), f32)
    zeros_blk = jnp.zeros((RPS, 128), f32)

    dsts_both = jnp.concatenate([dsts0, dsts1], axis=0)
    cnt_all = _sc_counts(dsts_both, ones_blk, zeros_blk)
    cnt0 = cnt_all[:NPAD, :16]
    cnt1 = cnt_all[NPAD:, :16]

    xp = jnp.pad(x.astype(f32), ((0, NPAD - N), (0, 0)))
    x1p = jnp.pad(x1.astype(f32), ((0, NPAD - N), (0, 0)))
    xs0 = _tc_scale(xp, cnt0)
    xs1 = _tc_scale(x1p, cnt1)

    s0 = _sc_segsum(xs0.reshape(2 * NPAD, 128), srcs0, dsts0, zeros_blk)
    s1 = _sc_segsum(xs1.reshape(2 * NPAD, 128), srcs1, dsts1, zeros_blk)

    ya, yb, ra, rb = _tc_dense(
        s0.reshape(2, NPAD, 128), xs0, s1.reshape(2, NPAD, 128), xs1,
        cnt0, cnt1, gcn1_W, gcn1_b.reshape(1, H), gcn2_W, gcn2_b.reshape(1, H),
        sage1_Wl, sage1_Wr, sage1_bl.reshape(1, K),
        sage2_Wl, sage2_Wr, sage2_bl.reshape(1, K))

    sa = _sc_segsum(ya.reshape(2 * NPAD, 128), srcs0, dsts0, zeros_blk)
    sb = _sc_segsum(yb.reshape(2 * NPAD, 128), srcs1, dsts1, zeros_blk)

    scaled1 = (kan1_spline_w * kan1_scaler[:, :, None]).reshape(K, K * 8)
    sc1 = scaled1.reshape(K, K, 8).transpose(2, 1, 0)
    w2 = (kan2_spline_w * kan2_scaler[:, :, None]).reshape(K, 8).T

    return _tc_kan(sa.reshape(2, NPAD, 128), sb.reshape(2, NPAD, 128),
                   ra, rb, cnt0, cnt1, kan1_base_w.T, sc1, kan2_base_w, w2)


# speedup vs baseline: 13.0043x; 1.1545x over previous
import functools

import jax
import jax.numpy as jnp
import numpy as np
from jax import lax
from jax.experimental import pallas as pl
from jax.experimental.pallas import tpu as pltpu
from jax.experimental.pallas import tpu_sc as plsc

N = 10000
E = 160000
D_IN = 256
H = 512
K = H // 2
GRID_SIZE = 5
SPLINE_ORDER = 3

NPAD = 10240
NTRASH = 240
CHUNK = 128
EPC = 1280
EP = EPC * CHUNK
CPS = EPC // 16
NBUF = 2
RPS = NPAD // 16
BLK = 1024
NBLK = NPAD // BLK

_G = (np.arange(-SPLINE_ORDER, GRID_SIZE + SPLINE_ORDER + 1, dtype=np.float32)
      * np.float32(2.0 / GRID_SIZE) - np.float32(1.0))

def _sc_mesh():
    return plsc.VectorSubcoreMesh(core_axis_name="c", subcore_axis_name="s")


@jax.jit
def _sc_counts(dsts2, ones_blk, zeros_blk):
    @functools.partial(
        pl.kernel,
        out_type=jax.ShapeDtypeStruct((2 * NPAD, 128), jnp.float32),
        mesh=_sc_mesh(),
        scratch_types=[
            pltpu.VMEM((CPS, CHUNK), jnp.int32),
            pltpu.VMEM((CHUNK, 128), jnp.float32),
            pltpu.VMEM_SHARED((NPAD, 128), jnp.float32),
            pltpu.SemaphoreType.DMA,
        ],
    )
    def kern(dsts_hbm, ones_hbm, zeros_hbm, out_hbm, dst_all, ones_v, acc, sem):
        c = lax.axis_index("c")
        s = lax.axis_index("s")
        pltpu.sync_copy(zeros_hbm, acc.at[pl.ds(s * RPS, RPS)])
        pltpu.sync_copy(ones_hbm, ones_v)
        pltpu.sync_copy(dsts_hbm.at[pl.ds(c * EPC + s * CPS, CPS)], dst_all)
        plsc.subcore_barrier()

        @pl.loop(0, CPS)
        def _(j):
            pltpu.sync_copy(ones_v, acc.at[dst_all.at[j]], add=True)

        plsc.subcore_barrier()
        pltpu.sync_copy(acc.at[pl.ds(s * RPS, RPS)],
                        out_hbm.at[pl.ds(c * NPAD + s * RPS, RPS)])

    return kern(dsts2, ones_blk, zeros_blk)


@jax.jit
def _sc_segsum(table, srcs2, dsts, zeros_blk):
    @functools.partial(
        pl.kernel,
        out_type=jax.ShapeDtypeStruct((2 * NPAD, 128), jnp.float32),
        mesh=_sc_mesh(),
        scratch_types=[
            pltpu.VMEM((CPS, CHUNK), jnp.int32),
        ] + [pltpu.VMEM((CHUNK,), jnp.int32) for _ in range(NBUF)]
          + [pltpu.VMEM((CHUNK, 128), jnp.float32) for _ in range(NBUF)]
          + [pltpu.SemaphoreType.DMA for _ in range(NBUF)]
          + [pltpu.VMEM_SHARED((NPAD, 128), jnp.float32)],
    )
    def kern(table_hbm, srcs_hbm, dsts_hbm, zeros_hbm, out_hbm,
             dst_all, *rest):
        src_v = rest[:NBUF]
        rows = rest[NBUF:2 * NBUF]
        sems = rest[2 * NBUF:3 * NBUF]
        acc = rest[3 * NBUF]
        c = lax.axis_index("c")
        s = lax.axis_index("s")
        pltpu.sync_copy(zeros_hbm, acc.at[pl.ds(s * RPS, RPS)])
        pltpu.sync_copy(dsts_hbm.at[pl.ds(s * CPS, CPS)], dst_all)
        plsc.subcore_barrier()

        for b in range(NBUF):
            pltpu.sync_copy(srcs_hbm.at[c * EPC + s * CPS + b], src_v[b])
            pltpu.async_copy(table_hbm.at[src_v[b]], rows[b], sems[b])

        @pl.loop(0, CPS // NBUF)
        def _(i):
            for b in range(NBUF):
                j = i * NBUF + b
                pltpu.make_async_copy(
                    table_hbm.at[src_v[b]], rows[b], sems[b]).wait()
                pltpu.sync_copy(rows[b], acc.at[dst_all.at[j]], add=True)

                @pl.when(j + NBUF < CPS)
                def _():
                    pltpu.sync_copy(
                        srcs_hbm.at[c * EPC + s * CPS + j + NBUF], src_v[b])
                    pltpu.async_copy(table_hbm.at[src_v[b]], rows[b], sems[b])

        plsc.subcore_barrier()
        pltpu.sync_copy(acc.at[pl.ds(s * RPS, RPS)],
                        out_hbm.at[pl.ds(c * NPAD + s * RPS, RPS)])

    return kern(table, srcs2, dsts, zeros_blk)


def _scale_body(x_ref, cnt_ref, out_ref):
    dinv = lax.rsqrt(cnt_ref[:, 0:1] + 1.0)
    xs = x_ref[...] * dinv
    out_ref[0] = xs[:, :128]
    out_ref[1] = xs[:, 128:]


@jax.jit
def _tc_scale(xp, cnt):
    return pl.pallas_call(
        _scale_body,
        grid=(NBLK,),
        in_specs=[
            pl.BlockSpec((BLK, 256), lambda i: (i, 0)),
            pl.BlockSpec((BLK, 16), lambda i: (i, 0)),
        ],
        out_specs=pl.BlockSpec((2, BLK, 128), lambda i: (0, i, 0)),
        out_shape=jax.ShapeDtypeStruct((2, NPAD, 128), jnp.float32),
    )(xp, cnt)


def _cat_halves(st):
    return jnp.concatenate([st[0], st[1]], axis=1)


def _dot(a, b):
    return jnp.dot(a, b, precision=jax.lax.Precision.HIGHEST)


def _dot_t(a, w):
    return lax.dot_general(a, w, (((1,), (1,)), ((), ())),
                           precision=jax.lax.Precision.HIGHEST)


def _dense_body(s0_ref, xs0_ref, s1_ref, xs1_ref, cnt0_ref, cnt1_ref,
                g1w_ref, g1b_ref, g2w_ref, g2b_ref,
                s1wl_ref, s1wr_ref, s1bl_ref, s2wl_ref, s2wr_ref, s2bl_ref,
                ya_ref, yb_ref, ra_ref, rb_ref):
    dinv0 = lax.rsqrt(cnt0_ref[:, 0:1] + 1.0)
    dinv1 = lax.rsqrt(cnt1_ref[:, 0:1] + 1.0)
    agg0 = dinv0 * (_cat_halves(s0_ref[...]) + _cat_halves(xs0_ref[...]))
    agg1 = dinv1 * (_cat_halves(s1_ref[...]) + _cat_halves(xs1_ref[...]))
    h0a = jax.nn.relu(_dot(agg0, g1w_ref[...]) + g1b_ref[...])
    h1c = _dot(agg1, g2w_ref[...]) + g2b_ref[...]
    h1 = jax.nn.relu(h1c)
    h0 = h0a + h1
    bpre = h1 + h1c

    ya = _dot_t(h0, s1wl_ref[...])
    yb = _dot_t(bpre, s2wl_ref[...])
    ya_ref[0], ya_ref[1] = ya[:, :128], ya[:, 128:]
    yb_ref[0], yb_ref[1] = yb[:, :128], yb[:, 128:]
    ra_ref[...] = _dot_t(h0, s1wr_ref[...]) + s1bl_ref[...]
    rb_ref[...] = _dot_t(bpre, s2wr_ref[...]) + s2bl_ref[...]


@jax.jit
def _tc_dense(s0, xs0, s1, xs1, cnt0, cnt1, g1w, g1b, g2w, g2b,
              s1wl, s1wr, s1bl, s2wl, s2wr, s2bl):
    st_spec = pl.BlockSpec((2, BLK, 128), lambda i: (0, i, 0))
    cnt_spec = pl.BlockSpec((BLK, 16), lambda i: (i, 0))
    full = lambda shape: pl.BlockSpec(shape, lambda i: tuple(0 for _ in shape))
    return pl.pallas_call(
        _dense_body,
        grid=(NBLK,),
        in_specs=[st_spec, st_spec, st_spec, st_spec, cnt_spec, cnt_spec,
                  full((256, 512)), full((1, 512)), full((256, 512)),
                  full((1, 512)),
                  full((256, 512)), full((256, 512)), full((1, 256)),
                  full((256, 512)), full((256, 512)), full((1, 256))],
        out_specs=[st_spec, st_spec,
                   pl.BlockSpec((BLK, 256), lambda i: (i, 0)),
                   pl.BlockSpec((BLK, 256), lambda i: (i, 0))],
        out_shape=[jax.ShapeDtypeStruct((2, NPAD, 128), jnp.float32),
                   jax.ShapeDtypeStruct((2, NPAD, 128), jnp.float32),
                   jax.ShapeDtypeStruct((NPAD, 256), jnp.float32),
                   jax.ShapeDtypeStruct((NPAD, 256), jnp.float32)],
    )(s0, xs0, s1, xs1, cnt0, cnt1, g1w, g1b, g2w, g2b,
      s1wl, s1wr, s1bl, s2wl, s2wr, s2bl)


def _silu(x):
    return x / (1.0 + jnp.exp(-x))


def _basis8(x):
    b = [jnp.where((x >= _G[j]) & (x < _G[j + 1]), 1.0, 0.0) for j in range(11)]
    for k in range(1, SPLINE_ORDER + 1):
        nb = []
        for j in range(11 - k):
            linv = np.float32(1.0) / (_G[j + k] - _G[j])
            rinv = np.float32(1.0) / (_G[j + k + 1] - _G[j + 1])
            nb.append((x - _G[j]) * linv * b[j]
                      + (_G[j + k + 1] - x) * rinv * b[j + 1])
        b = nb
    return b


def _kan_scalar(z, k1bw_ref, sc1_ref, k2bw_ref, w2_ref):
    b1 = _basis8(z)
    z1 = jnp.dot(_silu(z), k1bw_ref[...])
    for c in range(8):
        z1 = z1 + jnp.dot(b1[c], sc1_ref[c])
    b2 = _basis8(z1)
    out = jnp.sum(_silu(z1) * k2bw_ref[...], axis=1)
    for c in range(8):
        out = out + jnp.sum(b2[c] * w2_ref[c:c + 1, :], axis=1)
    return out


def _kan_body(sa_ref, sb_ref, ra_ref, rb_ref, cnt0_ref, cnt1_ref,
              k1bw_ref, sc1_ref, k2bw_ref, w2_ref, out_ref):
    i = pl.program_id(0)
    cnt0c = jnp.maximum(cnt0_ref[:, 0:1], 1.0)
    cnt1c = jnp.maximum(cnt1_ref[:, 0:1], 1.0)
    a = jax.nn.relu(_cat_halves(sa_ref[...]) / cnt0c + ra_ref[...])
    bb = jax.nn.relu(_cat_halves(sb_ref[...]) / cnt1c + rb_ref[...])
    va = _kan_scalar(a, k1bw_ref, sc1_ref, k2bw_ref, w2_ref)
    vb = _kan_scalar(bb, k1bw_ref, sc1_ref, k2bw_ref, w2_ref)
    rows = i * BLK + lax.broadcasted_iota(jnp.int32, (BLK,), 0)
    valid = rows < N
    part = jnp.sum(jnp.where(valid, va + vb, 0.0))

    @pl.when(i == 0)
    def _():
        out_ref[0, 0] = 0.0

    acc = out_ref[0, 0] + part

    @pl.when(i == NBLK - 1)
    def _():
        out_ref[0, 0] = acc / np.float32(2 * N)

    @pl.when(i < NBLK - 1)
    def _():
        out_ref[0, 0] = acc


@jax.jit
def _tc_kan(sa, sb, ra, rb, cnt0, cnt1, k1bw, sc1, k2bw, w2):
    st_spec = pl.BlockSpec((2, BLK, 128), lambda i: (0, i, 0))
    cnt_spec = pl.BlockSpec((BLK, 16), lambda i: (i, 0))
    full = lambda shape: pl.BlockSpec(shape, lambda i: tuple(0 for _ in shape))
    return pl.pallas_call(
        _kan_body,
        grid=(NBLK,),
        in_specs=[st_spec, st_spec,
                  pl.BlockSpec((BLK, 256), lambda i: (i, 0)),
                  pl.BlockSpec((BLK, 256), lambda i: (i, 0)),
                  cnt_spec, cnt_spec,
                  full((256, 256)), full((8, 256, 256)),
                  full((1, 256)), full((8, 256))],
        out_specs=pl.BlockSpec((1, 1), lambda i: (0, 0),
                               memory_space=pltpu.SMEM),
        out_shape=jax.ShapeDtypeStruct((1, 1), jnp.float32),
    )(sa, sb, ra, rb, cnt0, cnt1, k1bw, sc1, k2bw, w2)


def _prep_edges(edge_index):
    src = edge_index[0].astype(jnp.int32)
    dst = edge_index[1].astype(jnp.int32)
    pad_n = EP - E
    pad_src = (jnp.arange(pad_n, dtype=jnp.int32) % N)
    pad_dst = N + (jnp.arange(pad_n, dtype=jnp.int32) % NTRASH)
    srcp = jnp.concatenate([src, pad_src])
    dstp = jnp.concatenate([dst, pad_dst])
    srcs2 = jnp.concatenate([srcp, srcp + NPAD]).reshape(2 * EPC, CHUNK)
    dsts = dstp.reshape(EPC, CHUNK)
    return srcs2, dsts


def kernel(x, x1, edge_index, edge_index1, gcn1_W, gcn1_b, gcn2_W, gcn2_b,
           sage1_Wl, sage1_bl, sage1_Wr, sage2_Wl, sage2_bl, sage2_Wr,
           kan1_base_w, kan1_spline_w, kan1_scaler,
           kan2_base_w, kan2_spline_w, kan2_scaler):
    f32 = jnp.float32
    srcs0, dsts0 = _prep_edges(edge_index)
    srcs1, dsts1 = _prep_edges(edge_index1)

    ones_blk = jnp.ones((CHUNK, 128), f32)
    zeros_blk = jnp.zeros((RPS, 128), f32)

    dsts_both = jnp.concatenate([dsts0, dsts1], axis=0)
    cnt_all = _sc_counts(dsts_both, ones_blk, zeros_blk)
    cnt0 = cnt_all[:NPAD, :16]
    cnt1 = cnt_all[NPAD:, :16]

    xp = jnp.pad(x.astype(f32), ((0, NPAD - N), (0, 0)))
    x1p = jnp.pad(x1.astype(f32), ((0, NPAD - N), (0, 0)))
    xs0 = _tc_scale(xp, cnt0)
    xs1 = _tc_scale(x1p, cnt1)

    s0 = _sc_segsum(xs0.reshape(2 * NPAD, 128), srcs0, dsts0, zeros_blk)
    s1 = _sc_segsum(xs1.reshape(2 * NPAD, 128), srcs1, dsts1, zeros_blk)

    ya, yb, ra, rb = _tc_dense(
        s0.reshape(2, NPAD, 128), xs0, s1.reshape(2, NPAD, 128), xs1,
        cnt0, cnt1, gcn1_W, gcn1_b.reshape(1, H), gcn2_W, gcn2_b.reshape(1, H),
        sage1_Wl, sage1_Wr, sage1_bl.reshape(1, K),
        sage2_Wl, sage2_Wr, sage2_bl.reshape(1, K))

    sa = _sc_segsum(ya.reshape(2 * NPAD, 128), srcs0, dsts0, zeros_blk)
    sb = _sc_segsum(yb.reshape(2 * NPAD, 128), srcs1, dsts1, zeros_blk)

    scaled1 = (kan1_spline_w * kan1_scaler[:, :, None]).reshape(K, K * 8)
    sc1 = scaled1.reshape(K, K, 8).transpose(2, 1, 0)
    w2 = (kan2_spline_w * kan2_scaler[:, :, None]).reshape(K, 8).T

    return _tc_kan(sa.reshape(2, NPAD, 128), sb.reshape(2, NPAD, 128),
                   ra, rb, cnt0, cnt1, kan1_base_w.T, sc1, kan2_base_w, w2)


# speedup vs baseline: 13.8786x; 1.0672x over previous
import functools

import jax
import jax.numpy as jnp
import numpy as np
from jax import lax
from jax.experimental import pallas as pl
from jax.experimental.pallas import tpu as pltpu
from jax.experimental.pallas import tpu_sc as plsc

N = 10000
E = 160000
D_IN = 256
H = 512
K = H // 2
GRID_SIZE = 5
SPLINE_ORDER = 3

NPAD = 10240
NTRASH = 240
CHUNK = 128
EPC = 1280
EP = EPC * CHUNK
CPS = EPC // 16
NBUF = 2
RPS = NPAD // 16
BLK = 1024
NBLK = NPAD // BLK

_G = (np.arange(-SPLINE_ORDER, GRID_SIZE + SPLINE_ORDER + 1, dtype=np.float32)
      * np.float32(2.0 / GRID_SIZE) - np.float32(1.0))

def _sc_mesh():
    return plsc.VectorSubcoreMesh(core_axis_name="c", subcore_axis_name="s")


@jax.jit
def _sc_counts(dsts2, ones_blk, zeros_blk):
    @functools.partial(
        pl.kernel,
        out_type=jax.ShapeDtypeStruct((2 * NPAD, 128), jnp.float32),
        mesh=_sc_mesh(),
        scratch_types=[
            pltpu.VMEM((CPS, CHUNK), jnp.int32),
            pltpu.VMEM((CHUNK, 128), jnp.float32),
            pltpu.VMEM_SHARED((NPAD, 128), jnp.float32),
            pltpu.SemaphoreType.DMA,
        ],
    )
    def kern(dsts_hbm, ones_hbm, zeros_hbm, out_hbm, dst_all, ones_v, acc, sem):
        c = lax.axis_index("c")
        s = lax.axis_index("s")
        pltpu.sync_copy(zeros_hbm, acc.at[pl.ds(s * RPS, RPS)])
        pltpu.sync_copy(ones_hbm, ones_v)
        pltpu.sync_copy(dsts_hbm.at[pl.ds(c * EPC + s * CPS, CPS)], dst_all)
        plsc.subcore_barrier()

        @pl.loop(0, CPS)
        def _(j):
            pltpu.sync_copy(ones_v, acc.at[dst_all.at[j]], add=True)

        plsc.subcore_barrier()
        pltpu.sync_copy(acc.at[pl.ds(s * RPS, RPS)],
                        out_hbm.at[pl.ds(c * NPAD + s * RPS, RPS)])

    return kern(dsts2, ones_blk, zeros_blk)


@jax.jit
def _sc_segsum(table, srcs2, dsts, zeros_blk):
    @functools.partial(
        pl.kernel,
        out_type=jax.ShapeDtypeStruct((2 * NPAD, 128), jnp.float32),
        mesh=_sc_mesh(),
        scratch_types=[
            pltpu.VMEM((CPS, CHUNK), jnp.int32),
        ] + [pltpu.VMEM((CHUNK,), jnp.int32) for _ in range(NBUF)]
          + [pltpu.VMEM((CHUNK, 128), jnp.float32) for _ in range(NBUF)]
          + [pltpu.SemaphoreType.DMA for _ in range(NBUF)]
          + [pltpu.VMEM_SHARED((NPAD, 128), jnp.float32)],
    )
    def kern(table_hbm, srcs_hbm, dsts_hbm, zeros_hbm, out_hbm,
             dst_all, *rest):
        src_v = rest[:NBUF]
        rows = rest[NBUF:2 * NBUF]
        sems = rest[2 * NBUF:3 * NBUF]
        acc = rest[3 * NBUF]
        c = lax.axis_index("c")
        s = lax.axis_index("s")
        pltpu.sync_copy(zeros_hbm, acc.at[pl.ds(s * RPS, RPS)])
        pltpu.sync_copy(dsts_hbm.at[pl.ds(s * CPS, CPS)], dst_all)
        plsc.subcore_barrier()

        for b in range(NBUF):
            pltpu.sync_copy(srcs_hbm.at[c * EPC + s * CPS + b], src_v[b])
            pltpu.async_copy(table_hbm.at[src_v[b]], rows[b], sems[b])

        @pl.loop(0, CPS // NBUF)
        def _(i):
            for b in range(NBUF):
                j = i * NBUF + b
                pltpu.make_async_copy(
                    table_hbm.at[src_v[b]], rows[b], sems[b]).wait()
                pltpu.sync_copy(rows[b], acc.at[dst_all.at[j]], add=True)

                @pl.when(j + NBUF < CPS)
                def _():
                    pltpu.sync_copy(
                        srcs_hbm.at[c * EPC + s * CPS + j + NBUF], src_v[b])
                    pltpu.async_copy(table_hbm.at[src_v[b]], rows[b], sems[b])

        plsc.subcore_barrier()
        pltpu.sync_copy(acc.at[pl.ds(s * RPS, RPS)],
                        out_hbm.at[pl.ds(c * NPAD + s * RPS, RPS)])

    return kern(table, srcs2, dsts, zeros_blk)


def _scale_body(x_ref, cnt_ref, out_ref):
    dinv = lax.rsqrt(cnt_ref[:, 0:1] + 1.0)
    xs = x_ref[...] * dinv
    out_ref[0] = xs[:, :128]
    out_ref[1] = xs[:, 128:]


@jax.jit
def _tc_scale(xp, cnt):
    return pl.pallas_call(
        _scale_body,
        grid=(NBLK,),
        in_specs=[
            pl.BlockSpec((BLK, 256), lambda i: (i, 0)),
            pl.BlockSpec((BLK, 16), lambda i: (i, 0)),
        ],
        out_specs=pl.BlockSpec((2, BLK, 128), lambda i: (0, i, 0)),
        out_shape=jax.ShapeDtypeStruct((2, NPAD, 128), jnp.float32),
    )(xp, cnt)


def _cat_halves(st):
    return jnp.concatenate([st[0], st[1]], axis=1)


def _dot(a, b):
    return jnp.dot(a, b)


def _dot_t(a, w):
    return lax.dot_general(a, w, (((1,), (1,)), ((), ())))


def _bf(x):
    return x.astype(jnp.bfloat16).astype(jnp.float32)


def _dense_body(s0_ref, xs0_ref, s1_ref, xs1_ref, cnt0_ref, cnt1_ref,
                g1w_ref, g1b_ref, g2w_ref, g2b_ref,
                s1wl_ref, s1wr_ref, s1bl_ref, s2wl_ref, s2wr_ref, s2bl_ref,
                ya_ref, yb_ref, ra_ref, rb_ref):
    dinv0 = lax.rsqrt(cnt0_ref[:, 0:1] + 1.0)
    dinv1 = lax.rsqrt(cnt1_ref[:, 0:1] + 1.0)
    agg0 = dinv0 * (_cat_halves(s0_ref[...]) + _cat_halves(xs0_ref[...]))
    agg1 = dinv1 * (_cat_halves(s1_ref[...]) + _cat_halves(xs1_ref[...]))
    h0a = jax.nn.relu(_dot(agg0, g1w_ref[...]) + g1b_ref[...])
    h1c = _dot(agg1, g2w_ref[...]) + g2b_ref[...]
    h1 = jax.nn.relu(h1c)
    h0 = h0a + h1
    bpre = h1 + h1c

    ya = _dot_t(h0, s1wl_ref[...])
    yb = _dot_t(bpre, s2wl_ref[...])
    ya_ref[0], ya_ref[1] = ya[:, :128], ya[:, 128:]
    yb_ref[0], yb_ref[1] = yb[:, :128], yb[:, 128:]
    ra_ref[...] = _dot_t(h0, s1wr_ref[...]) + s1bl_ref[...]
    rb_ref[...] = _dot_t(bpre, s2wr_ref[...]) + s2bl_ref[...]


@jax.jit
def _tc_dense(s0, xs0, s1, xs1, cnt0, cnt1, g1w, g1b, g2w, g2b,
              s1wl, s1wr, s1bl, s2wl, s2wr, s2bl):
    st_spec = pl.BlockSpec((2, BLK, 128), lambda i: (0, i, 0))
    cnt_spec = pl.BlockSpec((BLK, 16), lambda i: (i, 0))
    full = lambda shape: pl.BlockSpec(shape, lambda i: tuple(0 for _ in shape))
    return pl.pallas_call(
        _dense_body,
        grid=(NBLK,),
        in_specs=[st_spec, st_spec, st_spec, st_spec, cnt_spec, cnt_spec,
                  full((256, 512)), full((1, 512)), full((256, 512)),
                  full((1, 512)),
                  full((256, 512)), full((256, 512)), full((1, 256)),
                  full((256, 512)), full((256, 512)), full((1, 256))],
        out_specs=[st_spec, st_spec,
                   pl.BlockSpec((BLK, 256), lambda i: (i, 0)),
                   pl.BlockSpec((BLK, 256), lambda i: (i, 0))],
        out_shape=[jax.ShapeDtypeStruct((2, NPAD, 128), jnp.float32),
                   jax.ShapeDtypeStruct((2, NPAD, 128), jnp.float32),
                   jax.ShapeDtypeStruct((NPAD, 256), jnp.float32),
                   jax.ShapeDtypeStruct((NPAD, 256), jnp.float32)],
    )(s0, xs0, s1, xs1, cnt0, cnt1, g1w, g1b, g2w, g2b,
      s1wl, s1wr, s1bl, s2wl, s2wr, s2bl)


def _silu(x):
    return x / (1.0 + jnp.exp(-x))


def _basis8(x):
    b = [jnp.where((x >= _G[j]) & (x < _G[j + 1]), 1.0, 0.0) for j in range(11)]
    for k in range(1, SPLINE_ORDER + 1):
        nb = []
        for j in range(11 - k):
            linv = np.float32(1.0) / (_G[j + k] - _G[j])
            rinv = np.float32(1.0) / (_G[j + k + 1] - _G[j + 1])
            nb.append((x - _G[j]) * linv * b[j]
                      + (_G[j + k + 1] - x) * rinv * b[j + 1])
        b = nb
    return b


def _kan_scalar(z, k1bw_ref, sc1_ref, k2bw_ref, w2_ref):
    b1 = _basis8(z)
    z1 = _dot(_silu(z), k1bw_ref[...])
    for c in range(8):
        z1 = z1 + _dot(b1[c], sc1_ref[c])
    b2 = _basis8(z1)
    out = jnp.sum(_bf(_silu(z1)) * _bf(k2bw_ref[...]), axis=1)
    for c in range(8):
        out = out + jnp.sum(_bf(b2[c]) * _bf(w2_ref[c:c + 1, :]), axis=1)
    return out


def _kan_body(sa_ref, sb_ref, ra_ref, rb_ref, cnt0_ref, cnt1_ref,
              k1bw_ref, sc1_ref, k2bw_ref, w2_ref, out_ref):
    i = pl.program_id(0)
    cnt0c = jnp.maximum(cnt0_ref[:, 0:1], 1.0)
    cnt1c = jnp.maximum(cnt1_ref[:, 0:1], 1.0)
    a = jax.nn.relu(_cat_halves(sa_ref[...]) / cnt0c + ra_ref[...])
    bb = jax.nn.relu(_cat_halves(sb_ref[...]) / cnt1c + rb_ref[...])
    va = _kan_scalar(a, k1bw_ref, sc1_ref, k2bw_ref, w2_ref)
    vb = _kan_scalar(bb, k1bw_ref, sc1_ref, k2bw_ref, w2_ref)
    rows = i * BLK + lax.broadcasted_iota(jnp.int32, (BLK,), 0)
    valid = rows < N
    part = jnp.sum(jnp.where(valid, va + vb, 0.0))

    @pl.when(i == 0)
    def _():
        out_ref[0, 0] = 0.0

    acc = out_ref[0, 0] + part

    @pl.when(i == NBLK - 1)
    def _():
        out_ref[0, 0] = acc / np.float32(2 * N)

    @pl.when(i < NBLK - 1)
    def _():
        out_ref[0, 0] = acc


@jax.jit
def _tc_kan(sa, sb, ra, rb, cnt0, cnt1, k1bw, sc1, k2bw, w2):
    st_spec = pl.BlockSpec((2, BLK, 128), lambda i: (0, i, 0))
    cnt_spec = pl.BlockSpec((BLK, 16), lambda i: (i, 0))
    full = lambda shape: pl.BlockSpec(shape, lambda i: tuple(0 for _ in shape))
    return pl.pallas_call(
        _kan_body,
        grid=(NBLK,),
        in_specs=[st_spec, st_spec,
                  pl.BlockSpec((BLK, 256), lambda i: (i, 0)),
                  pl.BlockSpec((BLK, 256), lambda i: (i, 0)),
                  cnt_spec, cnt_spec,
                  full((256, 256)), full((8, 256, 256)),
                  full((1, 256)), full((8, 256))],
        out_specs=pl.BlockSpec((1, 1), lambda i: (0, 0),
                               memory_space=pltpu.SMEM),
        out_shape=jax.ShapeDtypeStruct((1, 1), jnp.float32),
    )(sa, sb, ra, rb, cnt0, cnt1, k1bw, sc1, k2bw, w2)


def _prep_edges(edge_index):
    src = edge_index[0].astype(jnp.int32)
    dst = edge_index[1].astype(jnp.int32)
    pad_n = EP - E
    pad_src = (jnp.arange(pad_n, dtype=jnp.int32) % N)
    pad_dst = N + (jnp.arange(pad_n, dtype=jnp.int32) % NTRASH)
    srcp = jnp.concatenate([src, pad_src])
    dstp = jnp.concatenate([dst, pad_dst])
    srcs2 = jnp.concatenate([srcp, srcp + NPAD]).reshape(2 * EPC, CHUNK)
    dsts = dstp.reshape(EPC, CHUNK)
    return srcs2, dsts


def kernel(x, x1, edge_index, edge_index1, gcn1_W, gcn1_b, gcn2_W, gcn2_b,
           sage1_Wl, sage1_bl, sage1_Wr, sage2_Wl, sage2_bl, sage2_Wr,
           kan1_base_w, kan1_spline_w, kan1_scaler,
           kan2_base_w, kan2_spline_w, kan2_scaler):
    f32 = jnp.float32
    srcs0, dsts0 = _prep_edges(edge_index)
    srcs1, dsts1 = _prep_edges(edge_index1)

    ones_blk = jnp.ones((CHUNK, 128), f32)
    zeros_blk = jnp.zeros((RPS, 128), f32)

    dsts_both = jnp.concatenate([dsts0, dsts1], axis=0)
    cnt_all = _sc_counts(dsts_both, ones_blk, zeros_blk)
    cnt0 = cnt_all[:NPAD, :16]
    cnt1 = cnt_all[NPAD:, :16]

    xp = jnp.pad(x.astype(f32), ((0, NPAD - N), (0, 0)))
    x1p = jnp.pad(x1.astype(f32), ((0, NPAD - N), (0, 0)))
    xs0 = _tc_scale(xp, cnt0)
    xs1 = _tc_scale(x1p, cnt1)

    s0 = _sc_segsum(xs0.reshape(2 * NPAD, 128), srcs0, dsts0, zeros_blk)
    s1 = _sc_segsum(xs1.reshape(2 * NPAD, 128), srcs1, dsts1, zeros_blk)

    ya, yb, ra, rb = _tc_dense(
        s0.reshape(2, NPAD, 128), xs0, s1.reshape(2, NPAD, 128), xs1,
        cnt0, cnt1, gcn1_W, gcn1_b.reshape(1, H), gcn2_W, gcn2_b.reshape(1, H),
        sage1_Wl, sage1_Wr, sage1_bl.reshape(1, K),
        sage2_Wl, sage2_Wr, sage2_bl.reshape(1, K))

    sa = _sc_segsum(ya.reshape(2 * NPAD, 128), srcs0, dsts0, zeros_blk)
    sb = _sc_segsum(yb.reshape(2 * NPAD, 128), srcs1, dsts1, zeros_blk)

    scaled1 = (kan1_spline_w * kan1_scaler[:, :, None]).reshape(K, K * 8)
    sc1 = scaled1.reshape(K, K, 8).transpose(2, 1, 0)
    w2 = (kan2_spline_w * kan2_scaler[:, :, None]).reshape(K, 8).T

    return _tc_kan(sa.reshape(2, NPAD, 128), sb.reshape(2, NPAD, 128),
                   ra, rb, cnt0, cnt1, kan1_base_w.T, sc1, kan2_base_w, w2)
